# baseline (speedup 1.0000x reference)
import jax, jax.numpy as jnp
from jax import lax
from jax.experimental import pallas as pl
from jax.experimental.pallas import tpu as pltpu
from jax.experimental.pallas import tpu_sc as plsc

B, D = 2048, 16384
BPW = 64
NQ = 4
NU = (BPW // 8) * NQ
NBUF = 3
LA = 2
NC, NS = 2, 16


def _body(t4, idx_hbm, out4, idx_v, *rest):
    bufs = rest[:NBUF]
    in_sems, out_sems = rest[NBUF], rest[NBUF + 1]
    wid = lax.axis_index("s") * NC + lax.axis_index("c")
    base = wid * BPW

    pltpu.sync_copy(idx_hbm.at[pl.ds(base, BPW)], idx_v)
    rows = []
    for blk in range(BPW // 16):
        v = idx_v[pl.ds(blk * 16, 16)]
        rows.extend(v[j] for j in range(16))

    def start_gathers(u, b):
        band, q = divmod(u, NQ)
        for s in range(8):
            r = rows[band * 8 + s]
            pltpu.async_copy(
                t4.at[pl.ds(r // 8, 1), pl.ds(q * 32, 32), pl.ds(r % 8, 1), :],
                bufs[b].at[:, :, pl.ds(s, 1), :],
                in_sems.at[b],
            )

    def wait_gathers(u, b):
        for s in range(8):
            pltpu.make_async_copy(
                t4.at[pl.ds(0, 1), pl.ds(0, 32), pl.ds(0, 1), :],
                bufs[b].at[:, :, pl.ds(s, 1), :],
                in_sems.at[b],
            ).wait()

    def start_out(u, b):
        band, q = divmod(u, NQ)
        pltpu.async_copy(
            bufs[b],
            out4.at[pl.ds(wid * 8 + band, 1), pl.ds(q * 32, 32), :, :],
            out_sems.at[b],
        )

    def wait_out(b):
        pltpu.make_async_copy(
            bufs[b], out4.at[pl.ds(0, 1), pl.ds(0, 32), :, :], out_sems.at[b]
        ).wait()

    for h in range(LA):
        start_gathers(h, h % NBUF)
    for u in range(NU):
        b = u % NBUF
        wait_gathers(u, b)
        start_out(u, b)
        h = u + LA
        if h < NU:
            bh = h % NBUF
            if h >= NBUF:
                wait_out(bh)
            start_gathers(h, bh)
    for u in range(NU - NBUF, NU):
        wait_out(u % NBUF)


def kernel(prefix, table):
    idx = prefix.reshape(B)
    t4 = table.reshape(128, 8, 128, 128).transpose(0, 2, 1, 3)
    mesh = plsc.VectorSubcoreMesh(core_axis_name="c", subcore_axis_name="s",
                                  num_cores=NC, num_subcores=NS)
    f = pl.kernel(
        _body,
        out_type=jax.ShapeDtypeStruct((B // 8, 128, 8, 128), jnp.float32),
        mesh=mesh,
        scratch_types=(
            [pltpu.VMEM((BPW,), jnp.int32)]
            + [pltpu.VMEM((1, 32, 8, 128), jnp.float32) for _ in range(NBUF)]
            + [pltpu.SemaphoreType.DMA((NBUF,)),
               pltpu.SemaphoreType.DMA((NBUF,))]
        ),
    )
    out4 = f(t4, idx)
    return out4.transpose(0, 2, 1, 3).reshape(16, 128, D)
